# Initial kernel scaffold; baseline (speedup 1.0000x reference)
import jax
import jax.numpy as jnp
from jax.experimental import pallas as pl


def kernel(x, params, neigh, up_top, up_down):
    raise NotImplementedError("write your pallas kernel here")


# speedup vs baseline: 3.9255x; 3.9255x over previous
import jax
import jax.numpy as jnp
from jax import lax
from jax.experimental import pallas as pl
from jax.experimental.pallas import tpu as pltpu
from jax.experimental.pallas import tpu_sc as plsc

NCORE = 2
NSUB = 16
NW = NCORE * NSUB
LANES = 16
ALIGN = NW * LANES
_SMEM_WORDS = 1792

_MESH = plsc.VectorSubcoreMesh(core_axis_name="c", subcore_axis_name="s")


def _pad_to(n, m):
    return ((n + m - 1) // m) * m


def _wid():
    return lax.axis_index("s") * NCORE + lax.axis_index("c")


def _lrelu(v):
    return jnp.where(v >= 0, v, 0.2 * v)


def _bf16_round(v):
    u = plsc.bitcast(v, jnp.int32)
    r = u + 0x7FFF + ((u >> 16) & 1)
    return plsc.bitcast(r & jnp.int32(-65536), jnp.float32)


def _sc_conv(hflat, nb7, Wf, stf, N, ic, oc, Npad, xform):
    Bw = Npad // NW
    Gw = Bw // LANES

    budget = 131071 - (7 * Bw + oc * Bw + oc * 32 + 4096)
    cc = 1
    while (cc * 2 <= ic and (cc * 2) * Npad <= budget
           and 7 * (cc * 2) * (8 + 2 * oc) <= 5200):
        cc *= 2
    nchunks = ic // cc

    scratch = [
        pltpu.VMEM((7 * Bw,), jnp.int32),
        pltpu.VMEM((cc * Npad,), jnp.float32),
        pltpu.VMEM((oc * Bw,), jnp.float32),
        pltpu.VMEM((oc * 7 * ic * 16,), jnp.float32),
        pltpu.VMEM((2 * ic * 16,), jnp.float32),
        pltpu.VMEM((oc * 32,), jnp.float32),
    ]
    out_type = (
        jax.ShapeDtypeStruct((oc * Npad,), jnp.float32),
        jax.ShapeDtypeStruct((NW * oc * 32,), jnp.float32),
    )

    def body(h_hbm, nb_hbm, w_hbm, st_hbm, y_hbm, part_hbm,
             nbloc, hbuf, acc, wspl, stspl, statbuf):
        w = _wid()
        base = pl.multiple_of(w * Bw, LANES)
        pltpu.sync_copy(nb_hbm.at[pl.ds(pl.multiple_of(w * (7 * Bw), 8), 7 * Bw)],
                        nbloc)
        pltpu.sync_copy(w_hbm, wspl)
        if xform:
            pltpu.sync_copy(st_hbm, stspl)
        iota = lax.iota(jnp.int32, LANES)
        i7 = iota * 7
        zero = jnp.zeros((LANES,), jnp.float32)

        for chunk in range(nchunks):
            c0 = chunk * cc
            pltpu.sync_copy(h_hbm.at[pl.ds(c0 * Npad, cc * Npad)], hbuf)

            def gbody(g, _, chunk=chunk, c0=c0):
                regs = [zero] * oc
                for k in range(7):
                    pos = i7 + (g * (LANES * 7) + k)
                    nidx = plsc.load_gather(nbloc, [pos])
                    for ci in range(cc):
                        fidx = nidx + ci * Npad if ci else nidx
                        val = plsc.load_gather(hbuf, [fidx])
                        if xform:
                            sv = stspl[pl.ds((c0 + ci) * 16, 16)]
                            tv = stspl[pl.ds((ic + c0 + ci) * 16, 16)]
                            v2 = val * sv + tv
                            val = jnp.where(v2 >= 0, v2, 0.2 * v2)
                        val = _bf16_round(val)
                        for o in range(oc):
                            wv16 = wspl[pl.ds((o * 7 * ic + k * ic + c0 + ci) * 16, 16)]
                            regs[o] = regs[o] + wv16 * val
                for o in range(oc):
                    sl = pl.ds(o * Bw + g * LANES, LANES)
                    if chunk == 0:
                        acc[sl] = regs[o]
                    else:
                        plsc.addupdate(acc.at[sl], regs[o])
                return _

            lax.fori_loop(0, Gw, gbody, None)

        for o in range(oc):
            def sbody(g, carry, o=o):
                s_, q_ = carry
                v = acc[pl.ds(o * Bw + g * LANES, LANES)]
                gid = base + g * LANES + iota
                vm = jnp.where(gid < N, v, 0.0)
                return (s_ + vm, q_ + vm * vm)
            s_, q_ = lax.fori_loop(0, Gw, sbody, (zero, zero))
            statbuf[pl.ds(o * 32, LANES)] = s_
            statbuf[pl.ds(o * 32 + 16, LANES)] = q_

        for o in range(oc):
            pltpu.sync_copy(acc.at[pl.ds(o * Bw, Bw)],
                            y_hbm.at[pl.ds(o * Npad + base, Bw)])
        pltpu.sync_copy(statbuf, part_hbm.at[pl.ds(w * (oc * 32), oc * 32)])

    fn = pl.kernel(body, out_type=out_type, mesh=_MESH, scratch_types=scratch,
                   compiler_params=pltpu.CompilerParams(needs_layout_passes=False),
                   name=f"sc_conv_{ic}x{oc}_{Npad}")
    return fn(hflat, nb7, Wf, stf)


def _sc_gather(hflat, nb7, stf, ic, Npad, xform):
    Bw = Npad // NW
    Gw = Bw // LANES

    budget = 131071 - (7 * Bw + 4096)
    cc = 1
    while (cc * 2 <= ic and (cc * 2) * (Npad + 7 * Bw) <= budget
           and 7 * (cc * 2) * 10 <= 5200):
        cc *= 2
    nchunks = ic // cc

    scratch = [
        pltpu.VMEM((7 * Bw,), jnp.int32),
        pltpu.VMEM((cc * Npad,), jnp.float32),
        pltpu.VMEM((7 * cc * Bw,), jnp.float32),
        pltpu.VMEM((2 * ic * 16,), jnp.float32),
    ]
    out_type = jax.ShapeDtypeStruct((7 * ic * Npad,), jnp.float32)

    def body(h_hbm, nb_hbm, st_hbm, g_hbm, nbloc, hbuf, gbuf, stspl):
        w = _wid()
        base = pl.multiple_of(w * Bw, LANES)
        pltpu.sync_copy(nb_hbm.at[pl.ds(pl.multiple_of(w * (7 * Bw), 8), 7 * Bw)],
                        nbloc)
        if xform:
            pltpu.sync_copy(st_hbm, stspl)
        iota = lax.iota(jnp.int32, LANES)
        i7 = iota * 7

        for chunk in range(nchunks):
            c0 = chunk * cc
            pltpu.sync_copy(h_hbm.at[pl.ds(c0 * Npad, cc * Npad)], hbuf)

            def gbody(g, _, c0=c0):
                for k in range(7):
                    pos = i7 + (g * (LANES * 7) + k)
                    nidx = plsc.load_gather(nbloc, [pos])
                    for ci in range(cc):
                        fidx = nidx + ci * Npad if ci else nidx
                        val = plsc.load_gather(hbuf, [fidx])
                        if xform:
                            sv = stspl[pl.ds((c0 + ci) * 16, 16)]
                            tv = stspl[pl.ds((ic + c0 + ci) * 16, 16)]
                            v2 = val * sv + tv
                            val = jnp.where(v2 >= 0, v2, 0.2 * v2)
                        gbuf[pl.ds((k * cc + ci) * Bw + g * LANES, LANES)] = val
                return _

            lax.fori_loop(0, Gw, gbody, None)
            for k in range(7):
                for ci in range(cc):
                    pltpu.sync_copy(
                        gbuf.at[pl.ds((k * cc + ci) * Bw, Bw)],
                        g_hbm.at[pl.ds((k * ic + c0 + ci) * Npad + base, Bw)])

    fn = pl.kernel(body, out_type=out_type, mesh=_MESH, scratch_types=scratch,
                   compiler_params=pltpu.CompilerParams(needs_layout_passes=False),
                   name=f"sc_gather_{ic}_{Npad}")
    return fn(hflat, nb7, stf)


def _tc_convmm(gath2d, W, bng, bnb, N):
    kdim, Npad = gath2d.shape
    oc = W.shape[0]

    def body(g_ref, w_ref, bg_ref, bb_ref, y_ref, st_ref):
        gm = g_ref[...]
        y = lax.dot_general(w_ref[...], gm, (((1,), (0,)), ((), ())),
                            preferred_element_type=jnp.float32)
        y_ref[...] = y
        ids = lax.broadcasted_iota(jnp.int32, (1, Npad), 1)
        mf = (ids < N).astype(jnp.float32)
        ym = y * mf
        s = ym.sum(axis=1) / N
        q = (ym * ym).sum(axis=1) / N
        v = q - s * s
        sc = bg_ref[...] / jnp.sqrt(v + 1e-5)
        st_ref[0, :] = sc
        st_ref[1, :] = bb_ref[...] - s * sc

    return pl.pallas_call(
        body,
        out_shape=(jax.ShapeDtypeStruct((oc, Npad), jnp.float32),
                   jax.ShapeDtypeStruct((2, oc), jnp.float32)),
    )(gath2d, W, bng, bnb)


def _sc_pool(hflat, nbf7, stf, Nc, C, Nfpad, Ncpad):
    Bc = Ncpad // NW
    Gc = Bc // LANES

    budget = 131071 - (7 * Bc + 4096)
    cc = 1
    while cc * 2 <= C and (cc * 2) * (Nfpad + Bc) <= budget:
        cc *= 2
    nchunks = C // cc

    scratch = [
        pltpu.VMEM((7 * Bc,), jnp.int32),
        pltpu.VMEM((cc * Nfpad,), jnp.float32),
        pltpu.VMEM((cc * Bc,), jnp.float32),
        pltpu.VMEM((2 * C * 16,), jnp.float32),
    ]
    out_type = jax.ShapeDtypeStruct((C * Ncpad,), jnp.float32)

    def body(h_hbm, nb_hbm, st_hbm, o_hbm, nbloc, hbuf, outbuf, stspl):
        w = _wid()
        base = pl.multiple_of(w * Bc, LANES)
        pltpu.sync_copy(nb_hbm.at[pl.ds(pl.multiple_of(w * (7 * Bc), 8), 7 * Bc)],
                        nbloc)
        pltpu.sync_copy(st_hbm, stspl)
        iota = lax.iota(jnp.int32, LANES)
        i7 = iota * 7
        zero = jnp.zeros((LANES,), jnp.float32)
        inv7 = jnp.float32(1.0 / 7.0)

        for chunk in range(nchunks):
            c0 = chunk * cc
            pltpu.sync_copy(h_hbm.at[pl.ds(c0 * Nfpad, cc * Nfpad)], hbuf)

            def gbody(g, _, c0=c0):
                regs = [zero] * cc
                for k in range(7):
                    pos = i7 + (g * (LANES * 7) + k)
                    nidx = plsc.load_gather(nbloc, [pos])
                    for ci in range(cc):
                        fidx = nidx + ci * Nfpad if ci else nidx
                        val = plsc.load_gather(hbuf, [fidx])
                        sv = stspl[pl.ds((c0 + ci) * 16, 16)]
                        tv = stspl[pl.ds((C + c0 + ci) * 16, 16)]
                        v2 = val * sv + tv
                        val = jnp.where(v2 >= 0, v2, 0.2 * v2)
                        regs[ci] = regs[ci] + val
                for ci in range(cc):
                    outbuf[pl.ds(ci * Bc + g * LANES, LANES)] = regs[ci] * inv7
                return _

            lax.fori_loop(0, Gc, gbody, None)
            for ci in range(cc):
                pltpu.sync_copy(outbuf.at[pl.ds(ci * Bc, Bc)],
                                o_hbm.at[pl.ds((c0 + ci) * Ncpad + base, Bc)])

    fn = pl.kernel(body, out_type=out_type, mesh=_MESH, scratch_types=scratch,
                   compiler_params=pltpu.CompilerParams(needs_layout_passes=False),
                   name=f"sc_pool_{C}_{Ncpad}")
    return fn(hflat, nbf7, stf)


def _sc_assemble(y3r, jj1, jj2, skflat, skstf, Nc, oc, ocp, Nfpad):
    Bf = Nfpad // NW
    Gf = Bf // LANES

    scratch = [
        pltpu.VMEM((Bf,), jnp.int32),
        pltpu.VMEM((Bf,), jnp.int32),
        pltpu.VMEM((Bf, ocp), jnp.float32),
        pltpu.VMEM((Bf, ocp), jnp.float32),
        pltpu.VMEM((Bf,), jnp.float32),
        pltpu.VMEM((2 * oc * 16,), jnp.float32),
        pltpu.SemaphoreType.DMA,
    ]
    out_type = jax.ShapeDtypeStruct((2 * oc * Nfpad,), jnp.float32)

    def body(y_hbm, j1_hbm, j2_hbm, sk_hbm, st_hbm, o_hbm,
             j1loc, j2loc, rows1, rows2, ybuf, stspl, sem):
        w = _wid()
        base = pl.multiple_of(w * Bf, LANES)
        pltpu.sync_copy(j1_hbm.at[pl.ds(base, Bf)], j1loc)
        pltpu.sync_copy(j2_hbm.at[pl.ds(base, Bf)], j2loc)
        pltpu.sync_copy(st_hbm, stspl)
        descs = []
        q0 = 0
        while q0 < Bf:
            qn = min(128, Bf - q0)
            descs.append(pltpu.async_copy(
                y_hbm.at[j1loc.at[pl.ds(q0, qn)]], rows1.at[pl.ds(q0, qn)], sem))
            descs.append(pltpu.async_copy(
                y_hbm.at[j2loc.at[pl.ds(q0, qn)]], rows2.at[pl.ds(q0, qn)], sem))
            q0 += qn
        for d in descs:
            d.wait()

        iota = lax.iota(jnp.int32, LANES)
        for c in range(oc):
            rows = rows1 if c < oc // 2 else rows2
            q = 2 * c if c < oc // 2 else 2 * c - oc

            def gbody(g, _, c=c, rows=rows, q=q):
                fidx = iota + g * LANES
                m = (base + g * LANES + iota) < Nc
                cv = jnp.full((LANES,), c, jnp.int32)
                qv = jnp.full((LANES,), q, jnp.int32)
                col1 = jnp.where(m, cv, qv)
                col2 = jnp.where(m, cv, qv + 1)
                v1 = plsc.load_gather(rows, [fidx, col1])
                v2 = plsc.load_gather(rows, [fidx, col2])
                ybuf[pl.ds(g * LANES, LANES)] = (v1 + v2) * 0.5
                return _
            lax.fori_loop(0, Gf, gbody, None)
            pltpu.sync_copy(ybuf, o_hbm.at[pl.ds(c * Nfpad + base, Bf)])

        for c in range(oc):
            pltpu.sync_copy(sk_hbm.at[pl.ds(c * Nfpad + base, Bf)], ybuf)

            def tbody(g, _, c=c):
                sl = pl.ds(g * LANES, LANES)
                v = ybuf[sl]
                sv = stspl[pl.ds(c * 16, 16)]
                tv = stspl[pl.ds((oc + c) * 16, 16)]
                v2 = v * sv + tv
                ybuf[sl] = jnp.where(v2 >= 0, v2, 0.2 * v2)
                return _
            lax.fori_loop(0, Gf, tbody, None)
            pltpu.sync_copy(ybuf, o_hbm.at[pl.ds((oc + c) * Nfpad + base, Bf)])

    fn = pl.kernel(body, out_type=out_type, mesh=_MESH, scratch_types=scratch,
                   compiler_params=pltpu.CompilerParams(
                       needs_layout_passes=False, use_tc_tiling_on_sc=False),
                   name=f"sc_assemble_{oc}_{Nfpad}")
    return fn(y3r, jj1, jj2, skflat, skstf)


def _tc_upconv(hcm, st, Wp, bp):
    ic, Ncp = hcm.shape
    m = Wp.shape[0]

    def body(h_ref, st_ref, w_ref, b_ref, o_ref):
        x = h_ref[...]
        s = st_ref[0, :][:, None]
        t = st_ref[1, :][:, None]
        xn = _lrelu(x * s + t)
        z = lax.dot_general(xn, w_ref[...], (((0,), (1,)), ((), ())),
                            preferred_element_type=jnp.float32)
        o_ref[...] = z + b_ref[...][None, :]

    return pl.pallas_call(
        body,
        out_shape=jax.ShapeDtypeStruct((Ncp, m), jnp.float32),
    )(hcm, st, Wp, bp)


def _tc_final(hcm, st, W, b, N):
    ic, Npad = hcm.shape
    m = W.shape[0]
    B = 4096
    grid = (Npad + B - 1) // B

    def body(h_ref, st_ref, w_ref, b_ref, o_ref):
        x = h_ref[...]
        s = st_ref[0, :][:, None]
        t = st_ref[1, :][:, None]
        xn = _lrelu(x * s + t)
        z = lax.dot_general(xn, w_ref[...], (((0,), (1,)), ((), ())),
                            preferred_element_type=jnp.float32)
        o_ref[...] = z + b_ref[...][None, :]

    return pl.pallas_call(
        body,
        grid=(grid,),
        in_specs=[
            pl.BlockSpec((ic, B), lambda i: (0, i)),
            pl.BlockSpec((2, ic), lambda i: (0, 0)),
            pl.BlockSpec(W.shape, lambda i: (0, 0)),
            pl.BlockSpec(b.shape, lambda i: (0,)),
        ],
        out_specs=pl.BlockSpec((B, m), lambda i: (i, 0)),
        out_shape=jax.ShapeDtypeStruct((N, m), jnp.float32),
    )(hcm, st, W, b)


def _finalize(parts, g, be, N):
    oc = g.shape[0]
    p = parts.reshape(NW, oc, 2, 16)
    s_sum = p[:, :, 0, :].sum(axis=(0, 2))
    q_sum = p[:, :, 1, :].sum(axis=(0, 2))
    m = s_sum / N
    v = q_sum / N - m * m
    s = g / jnp.sqrt(v + 1e-5)
    t = be - m * s
    return jnp.stack([s, t])


def _pad1(a, npad):
    return jnp.pad(a, (0, npad - a.shape[0]))


def _splat16(a):
    return jnp.broadcast_to(a.reshape(-1)[:, None], (a.size, 16)).reshape(-1)


CHS = [2, 4, 8, 16, 32, 64]
LEVELS = [40962, 10242, 2562, 642, 162]


def _conv_layer(hflat, nb7, W, st, N, ic, Npad, bng, bnb):
    oc = W.shape[0]
    stf = (jnp.zeros((2 * ic * 16,), jnp.float32) if st is None
           else _splat16(st))
    if False:
        y, pt = _sc_conv(hflat, nb7, _splat16(W), stf, N, ic, oc, Npad,
                         st is not None)
        return y, _finalize(pt, bng, bnb, N)
    gath = _sc_gather(hflat, nb7, stf, ic, Npad, st is not None)
    y2d, st2 = _tc_convmm(gath.reshape(7 * ic, Npad), W, bng, bnb, N)
    return y2d.reshape(-1), st2


def kernel(x, params, neigh, up_top, up_down):
    NS = LEVELS
    npads = [_pad_to(n, ALIGN) for n in NS]
    nb_pad = [_pad1(neigh[i], 7 * npads[i]) for i in range(5)]

    h = jnp.pad(x.T, ((0, 0), (0, npads[0] - NS[0]))).reshape(-1)
    st = None
    chs = [CHS[i + 1] for i in range(5)]
    skips = []
    for i in range(5):
        p = params['down'][i]
        ic = CHS[i] if i == 0 else CHS[i]
        if i > 0:
            nbf = _pad1(neigh[i - 1][: NS[i] * 7], 7 * npads[i])
            h = _sc_pool(h, nbf, _splat16(st), NS[i], CHS[i],
                         npads[i - 1], npads[i])
            st = None
        y1, st1 = _conv_layer(h, nb_pad[i], p['c1W'], st, NS[i], CHS[i],
                              npads[i], p['bn1g'], p['bn1b'])
        y2, st = _conv_layer(y1, nb_pad[i], p['c2W'], st1, NS[i], CHS[i + 1],
                             npads[i], p['bn2g'], p['bn2b'])
        h = y2
        skips.append((y2, st))

    for i in range(4):
        p = params['up'][i]
        Nc, Nf = NS[4 - i], NS[3 - i]
        Ncp, Nfp = npads[4 - i], npads[3 - i]
        icu = CHS[5 - i]
        oc = p['c1W'].shape[0]
        ocp = max(16, oc)
        Wp = jnp.zeros((7, ocp, icu), jnp.float32)
        Wp = Wp.at[:, :oc, :].set(p['upW'].reshape(7, oc, icu))
        bp = jnp.zeros((7, ocp), jnp.float32).at[:, :oc].set(
            p['upb'].reshape(7, oc))
        y2d = _tc_upconv(h.reshape(icu, Ncp), st, Wp.reshape(7 * ocp, icu),
                         bp.reshape(7 * ocp))
        y3r = y2d.reshape(Ncp * 7, ocp)
        jj1 = _pad1(jnp.concatenate([up_top[i], up_down[i][0::2]]), Nfp)
        jj2 = _pad1(jnp.concatenate([up_top[i], up_down[i][1::2]]), Nfp)
        sk_raw, sk_st = skips[3 - i]
        hcat = _sc_assemble(y3r, jj1, jj2, sk_raw, _splat16(sk_st),
                            Nc, oc, ocp, Nfp)
        y1, st1 = _conv_layer(hcat, nb_pad[3 - i], p['c1W'], None, Nf,
                              2 * oc, Nfp, p['bn1g'], p['bn1b'])
        y2, st = _conv_layer(y1, nb_pad[3 - i], p['c2W'], st1, Nf,
                             oc, Nfp, p['bn2g'], p['bn2b'])
        h = y2

    return _tc_final(h.reshape(CHS[1], npads[0]), st,
                     params['outW'], params['outb'], NS[0])


# speedup vs baseline: 4.4818x; 1.1417x over previous
import jax
import jax.numpy as jnp
from jax import lax
from jax.experimental import pallas as pl
from jax.experimental.pallas import tpu as pltpu
from jax.experimental.pallas import tpu_sc as plsc

NCORE = 2
NSUB = 16
NW = NCORE * NSUB
LANES = 16
ALIGN = NW * LANES
_SMEM_WORDS = 1792

_MESH = plsc.VectorSubcoreMesh(core_axis_name="c", subcore_axis_name="s")


def _pad_to(n, m):
    return ((n + m - 1) // m) * m


def _wid():
    return lax.axis_index("s") * NCORE + lax.axis_index("c")


def _lrelu(v):
    return jnp.where(v >= 0, v, 0.2 * v)


def _bf16_round(v):
    u = plsc.bitcast(v, jnp.int32)
    r = u + 0x7FFF + ((u >> 16) & 1)
    return plsc.bitcast(r & jnp.int32(-65536), jnp.float32)


def _sc_conv(hflat, nb7, Wf, stf, N, ic, oc, Npad, xform):
    Bw = Npad // NW
    Gw = Bw // LANES

    budget = 131071 - (7 * Bw + oc * Bw + oc * 32 + 4096)
    cc = 1
    while (cc * 2 <= ic and (cc * 2) * Npad <= budget
           and 7 * (cc * 2) * (8 + 2 * oc) <= 5200):
        cc *= 2
    nchunks = ic // cc

    scratch = [
        pltpu.VMEM((7 * Bw,), jnp.int32),
        pltpu.VMEM((cc * Npad,), jnp.float32),
        pltpu.VMEM((oc * Bw,), jnp.float32),
        pltpu.VMEM((oc * 7 * ic * 16,), jnp.float32),
        pltpu.VMEM((2 * ic * 16,), jnp.float32),
        pltpu.VMEM((oc * 32,), jnp.float32),
    ]
    out_type = (
        jax.ShapeDtypeStruct((oc * Npad,), jnp.float32),
        jax.ShapeDtypeStruct((NW * oc * 32,), jnp.float32),
    )

    def body(h_hbm, nb_hbm, w_hbm, st_hbm, y_hbm, part_hbm,
             nbloc, hbuf, acc, wspl, stspl, statbuf):
        w = _wid()
        base = pl.multiple_of(w * Bw, LANES)
        pltpu.sync_copy(nb_hbm.at[pl.ds(pl.multiple_of(w * (7 * Bw), 8), 7 * Bw)],
                        nbloc)
        pltpu.sync_copy(w_hbm, wspl)
        if xform:
            pltpu.sync_copy(st_hbm, stspl)
        iota = lax.iota(jnp.int32, LANES)
        i7 = iota * 7
        zero = jnp.zeros((LANES,), jnp.float32)

        for chunk in range(nchunks):
            c0 = chunk * cc
            pltpu.sync_copy(h_hbm.at[pl.ds(c0 * Npad, cc * Npad)], hbuf)

            def gbody(g, _, chunk=chunk, c0=c0):
                regs = [zero] * oc
                for k in range(7):
                    pos = i7 + (g * (LANES * 7) + k)
                    nidx = plsc.load_gather(nbloc, [pos])
                    for ci in range(cc):
                        fidx = nidx + ci * Npad if ci else nidx
                        val = plsc.load_gather(hbuf, [fidx])
                        if xform:
                            sv = stspl[pl.ds((c0 + ci) * 16, 16)]
                            tv = stspl[pl.ds((ic + c0 + ci) * 16, 16)]
                            v2 = val * sv + tv
                            val = jnp.where(v2 >= 0, v2, 0.2 * v2)
                        val = _bf16_round(val)
                        for o in range(oc):
                            wv16 = wspl[pl.ds((o * 7 * ic + k * ic + c0 + ci) * 16, 16)]
                            regs[o] = regs[o] + wv16 * val
                for o in range(oc):
                    sl = pl.ds(o * Bw + g * LANES, LANES)
                    if chunk == 0:
                        acc[sl] = regs[o]
                    else:
                        plsc.addupdate(acc.at[sl], regs[o])
                return _

            lax.fori_loop(0, Gw, gbody, None)

        for o in range(oc):
            def sbody(g, carry, o=o):
                s_, q_ = carry
                v = acc[pl.ds(o * Bw + g * LANES, LANES)]
                gid = base + g * LANES + iota
                vm = jnp.where(gid < N, v, 0.0)
                return (s_ + vm, q_ + vm * vm)
            s_, q_ = lax.fori_loop(0, Gw, sbody, (zero, zero))
            statbuf[pl.ds(o * 32, LANES)] = s_
            statbuf[pl.ds(o * 32 + 16, LANES)] = q_

        for o in range(oc):
            pltpu.sync_copy(acc.at[pl.ds(o * Bw, Bw)],
                            y_hbm.at[pl.ds(o * Npad + base, Bw)])
        pltpu.sync_copy(statbuf, part_hbm.at[pl.ds(w * (oc * 32), oc * 32)])

    fn = pl.kernel(body, out_type=out_type, mesh=_MESH, scratch_types=scratch,
                   compiler_params=pltpu.CompilerParams(needs_layout_passes=False),
                   name=f"sc_conv_{ic}x{oc}_{Npad}")
    return fn(hflat, nb7, Wf, stf)


def _sc_gather(hflat, nb7, stf, ic, Npad, xform):
    Bw = Npad // NW
    Gw = Bw // LANES

    budget = 131071 - (7 * Bw + 4096)
    cc = 1
    while (cc * 2 <= ic and (cc * 2) * (Npad + 7 * Bw) <= budget
           and 7 * (cc * 2) * 10 <= 5200):
        cc *= 2
    nchunks = ic // cc

    scratch = [
        pltpu.VMEM((7 * Bw,), jnp.int32),
        pltpu.VMEM((cc * Npad,), jnp.float32),
        pltpu.VMEM((7 * cc * Bw,), jnp.float32),
        pltpu.VMEM((2 * ic * 16,), jnp.float32),
        pltpu.SemaphoreType.DMA,
    ]
    out_type = jax.ShapeDtypeStruct((7 * ic * Npad,), jnp.float32)

    def body(h_hbm, nb_hbm, st_hbm, g_hbm, nbloc, hbuf, gbuf, stspl, sem):
        w = _wid()
        base = pl.multiple_of(w * Bw, LANES)
        pltpu.sync_copy(nb_hbm.at[pl.ds(pl.multiple_of(w * (7 * Bw), 8), 7 * Bw)],
                        nbloc)
        if xform:
            pltpu.sync_copy(st_hbm, stspl)
        iota = lax.iota(jnp.int32, LANES)
        i7 = iota * 7

        descs = []
        for chunk in range(nchunks):
            c0 = chunk * cc
            pltpu.sync_copy(h_hbm.at[pl.ds(c0 * Npad, cc * Npad)], hbuf)
            for d in descs:
                d.wait()
            descs = []

            def gbody(g, _, c0=c0):
                for k in range(7):
                    pos = i7 + (g * (LANES * 7) + k)
                    nidx = plsc.load_gather(nbloc, [pos])
                    for ci in range(cc):
                        fidx = nidx + ci * Npad if ci else nidx
                        val = plsc.load_gather(hbuf, [fidx])
                        if xform:
                            sv = stspl[pl.ds((c0 + ci) * 16, 16)]
                            tv = stspl[pl.ds((ic + c0 + ci) * 16, 16)]
                            v2 = val * sv + tv
                            val = jnp.where(v2 >= 0, v2, 0.2 * v2)
                        gbuf[pl.ds((k * cc + ci) * Bw + g * LANES, LANES)] = val
                return _

            lax.fori_loop(0, Gw, gbody, None)
            for k in range(7):
                for ci in range(cc):
                    descs.append(pltpu.async_copy(
                        gbuf.at[pl.ds((k * cc + ci) * Bw, Bw)],
                        g_hbm.at[pl.ds((k * ic + c0 + ci) * Npad + base, Bw)],
                        sem))
        for d in descs:
            d.wait()

    fn = pl.kernel(body, out_type=out_type, mesh=_MESH, scratch_types=scratch,
                   compiler_params=pltpu.CompilerParams(needs_layout_passes=False),
                   name=f"sc_gather_{ic}_{Npad}")
    return fn(hflat, nb7, stf)


def _tc_convmm(gath2d, W, bng, bnb, N):
    kdim, Npad = gath2d.shape
    oc = W.shape[0]

    def body(g_ref, w_ref, bg_ref, bb_ref, y_ref, st_ref):
        gm = g_ref[...]
        y = lax.dot_general(w_ref[...], gm, (((1,), (0,)), ((), ())),
                            preferred_element_type=jnp.float32)
        y_ref[...] = y
        ids = lax.broadcasted_iota(jnp.int32, (1, Npad), 1)
        mf = (ids < N).astype(jnp.float32)
        ym = y * mf
        s = ym.sum(axis=1) / N
        q = (ym * ym).sum(axis=1) / N
        v = q - s * s
        sc = bg_ref[...] / jnp.sqrt(v + 1e-5)
        st_ref[0, :] = sc
        st_ref[1, :] = bb_ref[...] - s * sc

    return pl.pallas_call(
        body,
        out_shape=(jax.ShapeDtypeStruct((oc, Npad), jnp.float32),
                   jax.ShapeDtypeStruct((2, oc), jnp.float32)),
    )(gath2d, W, bng, bnb)


def _sc_pool(hflat, nbf7, stf, Nc, C, Nfpad, Ncpad):
    Bc = Ncpad // NW
    Gc = Bc // LANES

    budget = 131071 - (7 * Bc + 4096)
    cc = 1
    while cc * 2 <= C and (cc * 2) * (Nfpad + Bc) <= budget:
        cc *= 2
    nchunks = C // cc

    scratch = [
        pltpu.VMEM((7 * Bc,), jnp.int32),
        pltpu.VMEM((cc * Nfpad,), jnp.float32),
        pltpu.VMEM((cc * Bc,), jnp.float32),
        pltpu.VMEM((2 * C * 16,), jnp.float32),
    ]
    out_type = jax.ShapeDtypeStruct((C * Ncpad,), jnp.float32)

    def body(h_hbm, nb_hbm, st_hbm, o_hbm, nbloc, hbuf, outbuf, stspl):
        w = _wid()
        base = pl.multiple_of(w * Bc, LANES)
        pltpu.sync_copy(nb_hbm.at[pl.ds(pl.multiple_of(w * (7 * Bc), 8), 7 * Bc)],
                        nbloc)
        pltpu.sync_copy(st_hbm, stspl)
        iota = lax.iota(jnp.int32, LANES)
        i7 = iota * 7
        zero = jnp.zeros((LANES,), jnp.float32)
        inv7 = jnp.float32(1.0 / 7.0)

        for chunk in range(nchunks):
            c0 = chunk * cc
            pltpu.sync_copy(h_hbm.at[pl.ds(c0 * Nfpad, cc * Nfpad)], hbuf)

            def gbody(g, _, c0=c0):
                regs = [zero] * cc
                for k in range(7):
                    pos = i7 + (g * (LANES * 7) + k)
                    nidx = plsc.load_gather(nbloc, [pos])
                    for ci in range(cc):
                        fidx = nidx + ci * Nfpad if ci else nidx
                        val = plsc.load_gather(hbuf, [fidx])
                        sv = stspl[pl.ds((c0 + ci) * 16, 16)]
                        tv = stspl[pl.ds((C + c0 + ci) * 16, 16)]
                        v2 = val * sv + tv
                        val = jnp.where(v2 >= 0, v2, 0.2 * v2)
                        regs[ci] = regs[ci] + val
                for ci in range(cc):
                    outbuf[pl.ds(ci * Bc + g * LANES, LANES)] = regs[ci] * inv7
                return _

            lax.fori_loop(0, Gc, gbody, None)
            for ci in range(cc):
                pltpu.sync_copy(outbuf.at[pl.ds(ci * Bc, Bc)],
                                o_hbm.at[pl.ds((c0 + ci) * Ncpad + base, Bc)])

    fn = pl.kernel(body, out_type=out_type, mesh=_MESH, scratch_types=scratch,
                   compiler_params=pltpu.CompilerParams(needs_layout_passes=False),
                   name=f"sc_pool_{C}_{Ncpad}")
    return fn(hflat, nbf7, stf)


def _sc_assemble(y3r, jj1, jj2, skflat, skstf, Nc, oc, ocp, Nfpad):
    Bf = Nfpad // NW
    Gf = Bf // LANES

    scratch = [
        pltpu.VMEM((Bf,), jnp.int32),
        pltpu.VMEM((Bf,), jnp.int32),
        pltpu.VMEM((Bf, ocp), jnp.float32),
        pltpu.VMEM((Bf, ocp), jnp.float32),
        pltpu.VMEM((Bf,), jnp.float32),
        pltpu.VMEM((2 * oc * 16,), jnp.float32),
        pltpu.SemaphoreType.DMA,
    ]
    out_type = jax.ShapeDtypeStruct((2 * oc * Nfpad,), jnp.float32)

    def body(y_hbm, j1_hbm, j2_hbm, sk_hbm, st_hbm, o_hbm,
             j1loc, j2loc, rows1, rows2, ybuf, stspl, sem):
        w = _wid()
        base = pl.multiple_of(w * Bf, LANES)
        pltpu.sync_copy(j1_hbm.at[pl.ds(base, Bf)], j1loc)
        pltpu.sync_copy(j2_hbm.at[pl.ds(base, Bf)], j2loc)
        pltpu.sync_copy(st_hbm, stspl)
        descs = []
        q0 = 0
        while q0 < Bf:
            qn = min(128, Bf - q0)
            descs.append(pltpu.async_copy(
                y_hbm.at[j1loc.at[pl.ds(q0, qn)]], rows1.at[pl.ds(q0, qn)], sem))
            descs.append(pltpu.async_copy(
                y_hbm.at[j2loc.at[pl.ds(q0, qn)]], rows2.at[pl.ds(q0, qn)], sem))
            q0 += qn
        for d in descs:
            d.wait()

        iota = lax.iota(jnp.int32, LANES)
        for c in range(oc):
            rows = rows1 if c < oc // 2 else rows2
            q = 2 * c if c < oc // 2 else 2 * c - oc

            def gbody(g, _, c=c, rows=rows, q=q):
                fidx = iota + g * LANES
                m = (base + g * LANES + iota) < Nc
                cv = jnp.full((LANES,), c, jnp.int32)
                qv = jnp.full((LANES,), q, jnp.int32)
                col1 = jnp.where(m, cv, qv)
                col2 = jnp.where(m, cv, qv + 1)
                v1 = plsc.load_gather(rows, [fidx, col1])
                v2 = plsc.load_gather(rows, [fidx, col2])
                ybuf[pl.ds(g * LANES, LANES)] = (v1 + v2) * 0.5
                return _
            lax.fori_loop(0, Gf, gbody, None)
            pltpu.sync_copy(ybuf, o_hbm.at[pl.ds(c * Nfpad + base, Bf)])

        for c in range(oc):
            pltpu.sync_copy(sk_hbm.at[pl.ds(c * Nfpad + base, Bf)], ybuf)

            def tbody(g, _, c=c):
                sl = pl.ds(g * LANES, LANES)
                v = ybuf[sl]
                sv = stspl[pl.ds(c * 16, 16)]
                tv = stspl[pl.ds((oc + c) * 16, 16)]
                v2 = v * sv + tv
                ybuf[sl] = jnp.where(v2 >= 0, v2, 0.2 * v2)
                return _
            lax.fori_loop(0, Gf, tbody, None)
            pltpu.sync_copy(ybuf, o_hbm.at[pl.ds((oc + c) * Nfpad + base, Bf)])

    fn = pl.kernel(body, out_type=out_type, mesh=_MESH, scratch_types=scratch,
                   compiler_params=pltpu.CompilerParams(
                       needs_layout_passes=False, use_tc_tiling_on_sc=False),
                   name=f"sc_assemble_{oc}_{Nfpad}")
    return fn(y3r, jj1, jj2, skflat, skstf)


def _tc_upconv(hcm, st, Wp, bp):
    ic, Ncp = hcm.shape
    m = Wp.shape[0]

    def body(h_ref, st_ref, w_ref, b_ref, o_ref):
        x = h_ref[...]
        s = st_ref[0, :][:, None]
        t = st_ref[1, :][:, None]
        xn = _lrelu(x * s + t)
        z = lax.dot_general(xn, w_ref[...], (((0,), (1,)), ((), ())),
                            preferred_element_type=jnp.float32)
        o_ref[...] = z + b_ref[...][None, :]

    return pl.pallas_call(
        body,
        out_shape=jax.ShapeDtypeStruct((Ncp, m), jnp.float32),
    )(hcm, st, Wp, bp)


def _tc_final(hcm, st, W, b, N):
    ic, Npad = hcm.shape
    m = W.shape[0]
    B = 4096
    grid = (Npad + B - 1) // B

    def body(h_ref, st_ref, w_ref, b_ref, o_ref):
        x = h_ref[...]
        s = st_ref[0, :][:, None]
        t = st_ref[1, :][:, None]
        xn = _lrelu(x * s + t)
        z = lax.dot_general(xn, w_ref[...], (((0,), (1,)), ((), ())),
                            preferred_element_type=jnp.float32)
        o_ref[...] = z + b_ref[...][None, :]

    return pl.pallas_call(
        body,
        grid=(grid,),
        in_specs=[
            pl.BlockSpec((ic, B), lambda i: (0, i)),
            pl.BlockSpec((2, ic), lambda i: (0, 0)),
            pl.BlockSpec(W.shape, lambda i: (0, 0)),
            pl.BlockSpec(b.shape, lambda i: (0,)),
        ],
        out_specs=pl.BlockSpec((B, m), lambda i: (i, 0)),
        out_shape=jax.ShapeDtypeStruct((N, m), jnp.float32),
    )(hcm, st, W, b)


def _finalize(parts, g, be, N):
    oc = g.shape[0]
    p = parts.reshape(NW, oc, 2, 16)
    s_sum = p[:, :, 0, :].sum(axis=(0, 2))
    q_sum = p[:, :, 1, :].sum(axis=(0, 2))
    m = s_sum / N
    v = q_sum / N - m * m
    s = g / jnp.sqrt(v + 1e-5)
    t = be - m * s
    return jnp.stack([s, t])


def _pad1(a, npad):
    return jnp.pad(a, (0, npad - a.shape[0]))


def _splat16(a):
    return jnp.broadcast_to(a.reshape(-1)[:, None], (a.size, 16)).reshape(-1)


CHS = [2, 4, 8, 16, 32, 64]
LEVELS = [40962, 10242, 2562, 642, 162]


def _conv_layer(hflat, nb7, W, st, N, ic, Npad, bng, bnb):
    oc = W.shape[0]
    stf = (jnp.zeros((2 * ic * 16,), jnp.float32) if st is None
           else _splat16(st))
    if False:
        y, pt = _sc_conv(hflat, nb7, _splat16(W), stf, N, ic, oc, Npad,
                         st is not None)
        return y, _finalize(pt, bng, bnb, N)
    gath = _sc_gather(hflat, nb7, stf, ic, Npad, st is not None)
    y2d, st2 = _tc_convmm(gath.reshape(7 * ic, Npad), W, bng, bnb, N)
    return y2d.reshape(-1), st2


def kernel(x, params, neigh, up_top, up_down):
    NS = LEVELS
    npads = [_pad_to(n, ALIGN) for n in NS]
    nb_pad = [_pad1(neigh[i], 7 * npads[i]) for i in range(5)]

    h = jnp.pad(x.T, ((0, 0), (0, npads[0] - NS[0]))).reshape(-1)
    st = None
    chs = [CHS[i + 1] for i in range(5)]
    skips = []
    for i in range(5):
        p = params['down'][i]
        ic = CHS[i] if i == 0 else CHS[i]
        if i > 0:
            nbf = _pad1(neigh[i - 1][: NS[i] * 7], 7 * npads[i])
            h = _sc_pool(h, nbf, _splat16(st), NS[i], CHS[i],
                         npads[i - 1], npads[i])
            st = None
        y1, st1 = _conv_layer(h, nb_pad[i], p['c1W'], st, NS[i], CHS[i],
                              npads[i], p['bn1g'], p['bn1b'])
        y2, st = _conv_layer(y1, nb_pad[i], p['c2W'], st1, NS[i], CHS[i + 1],
                             npads[i], p['bn2g'], p['bn2b'])
        h = y2
        skips.append((y2, st))

    for i in range(4):
        p = params['up'][i]
        Nc, Nf = NS[4 - i], NS[3 - i]
        Ncp, Nfp = npads[4 - i], npads[3 - i]
        icu = CHS[5 - i]
        oc = p['c1W'].shape[0]
        ocp = max(16, oc)
        Wp = jnp.zeros((7, ocp, icu), jnp.float32)
        Wp = Wp.at[:, :oc, :].set(p['upW'].reshape(7, oc, icu))
        bp = jnp.zeros((7, ocp), jnp.float32).at[:, :oc].set(
            p['upb'].reshape(7, oc))
        y2d = _tc_upconv(h.reshape(icu, Ncp), st, Wp.reshape(7 * ocp, icu),
                         bp.reshape(7 * ocp))
        y3r = y2d.reshape(Ncp * 7, ocp)
        jj1 = _pad1(jnp.concatenate([up_top[i], up_down[i][0::2]]), Nfp)
        jj2 = _pad1(jnp.concatenate([up_top[i], up_down[i][1::2]]), Nfp)
        sk_raw, sk_st = skips[3 - i]
        hcat = _sc_assemble(y3r, jj1, jj2, sk_raw, _splat16(sk_st),
                            Nc, oc, ocp, Nfp)
        y1, st1 = _conv_layer(hcat, nb_pad[3 - i], p['c1W'], None, Nf,
                              2 * oc, Nfp, p['bn1g'], p['bn1b'])
        y2, st = _conv_layer(y1, nb_pad[3 - i], p['c2W'], st1, Nf,
                             oc, Nfp, p['bn2g'], p['bn2b'])
        h = y2

    return _tc_final(h.reshape(CHS[1], npads[0]), st,
                     params['outW'], params['outb'], NS[0])


# speedup vs baseline: 5.2541x; 1.1723x over previous
import jax
import jax.numpy as jnp
from jax import lax
from jax.experimental import pallas as pl
from jax.experimental.pallas import tpu as pltpu
from jax.experimental.pallas import tpu_sc as plsc

NCORE = 2
NSUB = 16
NW = NCORE * NSUB
LANES = 16
ALIGN = NW * LANES
_SMEM_WORDS = 1792

_MESH = plsc.VectorSubcoreMesh(core_axis_name="c", subcore_axis_name="s")


def _pad_to(n, m):
    return ((n + m - 1) // m) * m


def _wid():
    return lax.axis_index("s") * NCORE + lax.axis_index("c")


def _lrelu(v):
    return jnp.where(v >= 0, v, 0.2 * v)


def _sc_gather(hflat, nb7, stf, ic, Npad, xform):
    Bw = Npad // NW
    Gw = Bw // LANES

    budget = 131071 - (7 * Bw + 4096)
    cc = 1
    while (cc * 2 <= ic and (cc * 2) * (Npad + 7 * Bw) <= budget
           and 7 * (cc * 2) * 10 <= 5200):
        cc *= 2
    nchunks = ic // cc

    scratch = [
        pltpu.VMEM((7 * Bw,), jnp.int32),
        pltpu.VMEM((cc * Npad,), jnp.float32),
        pltpu.VMEM((7 * cc * Bw,), jnp.float32),
        pltpu.VMEM((2 * ic * 16,), jnp.float32),
        pltpu.VMEM_SHARED((ic * Npad,), jnp.float32),
        pltpu.SemaphoreType.DMA,
    ]
    out_type = jax.ShapeDtypeStruct((7 * ic * Npad,), jnp.float32)

    seg = ic * Npad // NSUB

    def body(h_hbm, nb_hbm, st_hbm, g_hbm, nbloc, hbuf, gbuf, stspl, sptab, sem):
        w = _wid()
        base = pl.multiple_of(w * Bw, LANES)
        soff = pl.multiple_of(lax.axis_index("s") * seg, 8)
        pre = [pltpu.async_copy(
            nb_hbm.at[pl.ds(pl.multiple_of(w * (7 * Bw), 8), 7 * Bw)], nbloc, sem)]
        if xform:
            pre.append(pltpu.async_copy(st_hbm, stspl, sem))
        pltpu.sync_copy(h_hbm.at[pl.ds(soff, seg)], hbuf.at[pl.ds(0, seg)])
        pltpu.sync_copy(hbuf.at[pl.ds(0, seg)], sptab.at[pl.ds(soff, seg)])
        for d in pre:
            d.wait()
        plsc.subcore_barrier()
        iota = lax.iota(jnp.int32, LANES)
        i7 = iota * 7

        descs = []
        for chunk in range(nchunks):
            c0 = chunk * cc
            pltpu.sync_copy(sptab.at[pl.ds(c0 * Npad, cc * Npad)], hbuf)
            for d in descs:
                d.wait()
            descs = []

            def gbody(g, _, c0=c0):
                for k in range(7):
                    pos = i7 + (g * (LANES * 7) + k)
                    nidx = plsc.load_gather(nbloc, [pos])
                    for ci in range(cc):
                        fidx = nidx + ci * Npad if ci else nidx
                        val = plsc.load_gather(hbuf, [fidx])
                        if xform:
                            sv = stspl[pl.ds((c0 + ci) * 16, 16)]
                            tv = stspl[pl.ds((ic + c0 + ci) * 16, 16)]
                            v2 = val * sv + tv
                            val = jnp.where(v2 >= 0, v2, 0.2 * v2)
                        gbuf[pl.ds((k * cc + ci) * Bw + g * LANES, LANES)] = val
                return _

            lax.fori_loop(0, Gw, gbody, None)
            for k in range(7):
                for ci in range(cc):
                    descs.append(pltpu.async_copy(
                        gbuf.at[pl.ds((k * cc + ci) * Bw, Bw)],
                        g_hbm.at[pl.ds((k * ic + c0 + ci) * Npad + base, Bw)],
                        sem))
        for d in descs:
            d.wait()

    fn = pl.kernel(body, out_type=out_type, mesh=_MESH, scratch_types=scratch,
                   compiler_params=pltpu.CompilerParams(needs_layout_passes=False),
                   name=f"sc_gather_{ic}_{Npad}")
    return fn(hflat, nb7, stf)


def _tc_convmm(gath2d, W, bng, bnb, N):
    kdim, Npad = gath2d.shape
    oc = W.shape[0]

    def body(g_ref, w_ref, bg_ref, bb_ref, y_ref, st_ref):
        gm = g_ref[...]
        y = lax.dot_general(w_ref[...], gm, (((1,), (0,)), ((), ())),
                            preferred_element_type=jnp.float32)
        y_ref[...] = y
        ids = lax.broadcasted_iota(jnp.int32, (1, Npad), 1)
        mf = (ids < N).astype(jnp.float32)
        ym = y * mf
        s = ym.sum(axis=1) / N
        q = (ym * ym).sum(axis=1) / N
        v = q - s * s
        sc = bg_ref[...] / jnp.sqrt(v + 1e-5)
        st_ref[0, :] = sc
        st_ref[1, :] = bb_ref[...] - s * sc

    return pl.pallas_call(
        body,
        out_shape=(jax.ShapeDtypeStruct((oc, Npad), jnp.float32),
                   jax.ShapeDtypeStruct((2, oc), jnp.float32)),
    )(gath2d, W, bng, bnb)


def _sc_pool(hflat, nbf7, stf, Nc, C, Nfpad, Ncpad):
    Bc = Ncpad // NW
    Gc = Bc // LANES

    budget = 131071 - (7 * Bc + 4096)
    cc = 1
    while cc * 2 <= C and (cc * 2) * (Nfpad + Bc) <= budget:
        cc *= 2
    nchunks = C // cc

    scratch = [
        pltpu.VMEM((7 * Bc,), jnp.int32),
        pltpu.VMEM((cc * Nfpad,), jnp.float32),
        pltpu.VMEM((cc * Bc,), jnp.float32),
        pltpu.VMEM((2 * C * 16,), jnp.float32),
        pltpu.VMEM_SHARED((C * Nfpad,), jnp.float32),
        pltpu.SemaphoreType.DMA,
    ]
    out_type = jax.ShapeDtypeStruct((C * Ncpad,), jnp.float32)

    seg = C * Nfpad // NSUB

    def body(h_hbm, nb_hbm, st_hbm, o_hbm, nbloc, hbuf, outbuf, stspl, sptab, sem):
        w = _wid()
        base = pl.multiple_of(w * Bc, LANES)
        soff = pl.multiple_of(lax.axis_index("s") * seg, 8)
        pre = [pltpu.async_copy(
            nb_hbm.at[pl.ds(pl.multiple_of(w * (7 * Bc), 8), 7 * Bc)], nbloc, sem),
            pltpu.async_copy(st_hbm, stspl, sem)]
        pltpu.sync_copy(h_hbm.at[pl.ds(soff, seg)], hbuf.at[pl.ds(0, seg)])
        pltpu.sync_copy(hbuf.at[pl.ds(0, seg)], sptab.at[pl.ds(soff, seg)])
        for d in pre:
            d.wait()
        plsc.subcore_barrier()
        iota = lax.iota(jnp.int32, LANES)
        i7 = iota * 7
        zero = jnp.zeros((LANES,), jnp.float32)
        inv7 = jnp.float32(1.0 / 7.0)

        descs = []
        for chunk in range(nchunks):
            c0 = chunk * cc
            pltpu.sync_copy(sptab.at[pl.ds(c0 * Nfpad, cc * Nfpad)], hbuf)
            for d in descs:
                d.wait()
            descs = []

            def gbody(g, _, c0=c0):
                regs = [zero] * cc
                for k in range(7):
                    pos = i7 + (g * (LANES * 7) + k)
                    nidx = plsc.load_gather(nbloc, [pos])
                    for ci in range(cc):
                        fidx = nidx + ci * Nfpad if ci else nidx
                        val = plsc.load_gather(hbuf, [fidx])
                        sv = stspl[pl.ds((c0 + ci) * 16, 16)]
                        tv = stspl[pl.ds((C + c0 + ci) * 16, 16)]
                        v2 = val * sv + tv
                        val = jnp.where(v2 >= 0, v2, 0.2 * v2)
                        regs[ci] = regs[ci] + val
                for ci in range(cc):
                    outbuf[pl.ds(ci * Bc + g * LANES, LANES)] = regs[ci] * inv7
                return _

            lax.fori_loop(0, Gc, gbody, None)
            for ci in range(cc):
                descs.append(pltpu.async_copy(
                    outbuf.at[pl.ds(ci * Bc, Bc)],
                    o_hbm.at[pl.ds((c0 + ci) * Ncpad + base, Bc)], sem))
        for d in descs:
            d.wait()

    fn = pl.kernel(body, out_type=out_type, mesh=_MESH, scratch_types=scratch,
                   compiler_params=pltpu.CompilerParams(needs_layout_passes=False),
                   name=f"sc_pool_{C}_{Ncpad}")
    return fn(hflat, nbf7, stf)


def _sc_assemble(y3r, jj1, jj2, skflat, skstf, Nc, oc, ocp, Nfpad):
    Bf = Nfpad // NW
    Gf = Bf // LANES

    scratch = [
        pltpu.VMEM((Bf,), jnp.int32),
        pltpu.VMEM((Bf,), jnp.int32),
        pltpu.VMEM((Bf, ocp), jnp.float32),
        pltpu.VMEM((Bf, ocp), jnp.float32),
        pltpu.VMEM((2 * Bf,), jnp.float32),
        pltpu.VMEM((2 * oc * 16,), jnp.float32),
        pltpu.SemaphoreType.DMA,
        pltpu.SemaphoreType.DMA,
    ]
    out_type = jax.ShapeDtypeStruct((2 * oc * Nfpad,), jnp.float32)

    def body(y_hbm, j1_hbm, j2_hbm, sk_hbm, st_hbm, o_hbm,
             j1loc, j2loc, rows1, rows2, ybuf, stspl, sem, sem2):
        w = _wid()
        base = pl.multiple_of(w * Bf, LANES)
        pltpu.sync_copy(j1_hbm.at[pl.ds(base, Bf)], j1loc)
        pltpu.sync_copy(j2_hbm.at[pl.ds(base, Bf)], j2loc)
        pltpu.sync_copy(st_hbm, stspl)
        descs = []
        q0 = 0
        while q0 < Bf:
            qn = min(128, Bf - q0)
            descs.append(pltpu.async_copy(
                y_hbm.at[j1loc.at[pl.ds(q0, qn)]], rows1.at[pl.ds(q0, qn)], sem))
            descs.append(pltpu.async_copy(
                y_hbm.at[j2loc.at[pl.ds(q0, qn)]], rows2.at[pl.ds(q0, qn)], sem))
            q0 += qn
        for d in descs:
            d.wait()

        iota = lax.iota(jnp.int32, LANES)
        outd = {}
        for c in range(oc):
            rows = rows1 if c < oc // 2 else rows2
            q = 2 * c if c < oc // 2 else 2 * c - oc
            half = (c % 2) * Bf
            if c >= 2:
                outd[c - 2].wait()

            def gbody(g, _, c=c, rows=rows, q=q, half=half):
                fidx = iota + g * LANES
                m = (base + g * LANES + iota) < Nc
                cv = jnp.full((LANES,), c, jnp.int32)
                qv = jnp.full((LANES,), q, jnp.int32)
                col1 = jnp.where(m, cv, qv)
                col2 = jnp.where(m, cv, qv + 1)
                v1 = plsc.load_gather(rows, [fidx, col1])
                v2 = plsc.load_gather(rows, [fidx, col2])
                ybuf[pl.ds(half + g * LANES, LANES)] = (v1 + v2) * 0.5
                return _
            lax.fori_loop(0, Gf, gbody, None)
            outd[c] = pltpu.async_copy(ybuf.at[pl.ds(half, Bf)],
                                       o_hbm.at[pl.ds(c * Nfpad + base, Bf)], sem)

        for c2 in range(oc):
            c = oc + c2
            half = (c % 2) * Bf
            if c >= 2:
                outd[c - 2].wait()
            pltpu.async_copy(sk_hbm.at[pl.ds(c2 * Nfpad + base, Bf)],
                             ybuf.at[pl.ds(half, Bf)], sem2).wait()

            def tbody(g, _, c2=c2, half=half):
                sl = pl.ds(half + g * LANES, LANES)
                v = ybuf[sl]
                sv = stspl[pl.ds(c2 * 16, 16)]
                tv = stspl[pl.ds((oc + c2) * 16, 16)]
                v2 = v * sv + tv
                ybuf[sl] = jnp.where(v2 >= 0, v2, 0.2 * v2)
                return _
            lax.fori_loop(0, Gf, tbody, None)
            outd[c] = pltpu.async_copy(ybuf.at[pl.ds(half, Bf)],
                                       o_hbm.at[pl.ds((oc + c2) * Nfpad + base, Bf)], sem)
        outd[2 * oc - 2].wait()
        outd[2 * oc - 1].wait()

    fn = pl.kernel(body, out_type=out_type, mesh=_MESH, scratch_types=scratch,
                   compiler_params=pltpu.CompilerParams(
                       needs_layout_passes=False, use_tc_tiling_on_sc=False),
                   name=f"sc_assemble_{oc}_{Nfpad}")
    return fn(y3r, jj1, jj2, skflat, skstf)


def _tc_upconv(hcm, st, Wp, bp):
    ic, Ncp = hcm.shape
    m = Wp.shape[0]

    def body(h_ref, st_ref, w_ref, b_ref, o_ref):
        x = h_ref[...]
        s = st_ref[0, :][:, None]
        t = st_ref[1, :][:, None]
        xn = _lrelu(x * s + t)
        z = lax.dot_general(xn, w_ref[...], (((0,), (1,)), ((), ())),
                            preferred_element_type=jnp.float32)
        o_ref[...] = z + b_ref[...][None, :]

    return pl.pallas_call(
        body,
        out_shape=jax.ShapeDtypeStruct((Ncp, m), jnp.float32),
    )(hcm, st, Wp, bp)


def _tc_final(hcm, st, W, b, N):
    ic, Npad = hcm.shape
    m = W.shape[0]
    B = 4096
    grid = (Npad + B - 1) // B

    def body(h_ref, st_ref, w_ref, b_ref, o_ref):
        x = h_ref[...]
        s = st_ref[0, :][:, None]
        t = st_ref[1, :][:, None]
        xn = _lrelu(x * s + t)
        z = lax.dot_general(xn, w_ref[...], (((0,), (1,)), ((), ())),
                            preferred_element_type=jnp.float32)
        o_ref[...] = z + b_ref[...][None, :]

    return pl.pallas_call(
        body,
        grid=(grid,),
        in_specs=[
            pl.BlockSpec((ic, B), lambda i: (0, i)),
            pl.BlockSpec((2, ic), lambda i: (0, 0)),
            pl.BlockSpec(W.shape, lambda i: (0, 0)),
            pl.BlockSpec(b.shape, lambda i: (0,)),
        ],
        out_specs=pl.BlockSpec((B, m), lambda i: (i, 0)),
        out_shape=jax.ShapeDtypeStruct((N, m), jnp.float32),
    )(hcm, st, W, b)


def _pad1(a, npad):
    return jnp.pad(a, (0, npad - a.shape[0]))


def _splat16(a):
    return jnp.broadcast_to(a.reshape(-1)[:, None], (a.size, 16)).reshape(-1)


CHS = [2, 4, 8, 16, 32, 64]
LEVELS = [40962, 10242, 2562, 642, 162]


def _conv_layer(hflat, nb7, W, st, N, ic, Npad, bng, bnb):
    oc = W.shape[0]
    stf = (jnp.zeros((2 * ic * 16,), jnp.float32) if st is None
           else _splat16(st))
    gath = _sc_gather(hflat, nb7, stf, ic, Npad, st is not None)
    y2d, st2 = _tc_convmm(gath.reshape(7 * ic, Npad), W, bng, bnb, N)
    return y2d.reshape(-1), st2


def kernel(x, params, neigh, up_top, up_down):
    NS = LEVELS
    npads = [_pad_to(n, ALIGN) for n in NS]
    nb_pad = [_pad1(neigh[i], 7 * npads[i]) for i in range(5)]

    h = jnp.pad(x.T, ((0, 0), (0, npads[0] - NS[0]))).reshape(-1)
    st = None
    chs = [CHS[i + 1] for i in range(5)]
    skips = []
    for i in range(5):
        p = params['down'][i]
        ic = CHS[i] if i == 0 else CHS[i]
        if i > 0:
            nbf = _pad1(neigh[i - 1][: NS[i] * 7], 7 * npads[i])
            h = _sc_pool(h, nbf, _splat16(st), NS[i], CHS[i],
                         npads[i - 1], npads[i])
            st = None
        y1, st1 = _conv_layer(h, nb_pad[i], p['c1W'], st, NS[i], CHS[i],
                              npads[i], p['bn1g'], p['bn1b'])
        y2, st = _conv_layer(y1, nb_pad[i], p['c2W'], st1, NS[i], CHS[i + 1],
                             npads[i], p['bn2g'], p['bn2b'])
        h = y2
        skips.append((y2, st))

    for i in range(4):
        p = params['up'][i]
        Nc, Nf = NS[4 - i], NS[3 - i]
        Ncp, Nfp = npads[4 - i], npads[3 - i]
        icu = CHS[5 - i]
        oc = p['c1W'].shape[0]
        ocp = max(16, oc)
        Wp = jnp.zeros((7, ocp, icu), jnp.float32)
        Wp = Wp.at[:, :oc, :].set(p['upW'].reshape(7, oc, icu))
        bp = jnp.zeros((7, ocp), jnp.float32).at[:, :oc].set(
            p['upb'].reshape(7, oc))
        y2d = _tc_upconv(h.reshape(icu, Ncp), st, Wp.reshape(7 * ocp, icu),
                         bp.reshape(7 * ocp))
        y3r = y2d.reshape(Ncp * 7, ocp)
        jj1 = _pad1(jnp.concatenate([up_top[i], up_down[i][0::2]]), Nfp)
        jj2 = _pad1(jnp.concatenate([up_top[i], up_down[i][1::2]]), Nfp)
        sk_raw, sk_st = skips[3 - i]
        hcat = _sc_assemble(y3r, jj1, jj2, sk_raw, _splat16(sk_st),
                            Nc, oc, ocp, Nfp)
        y1, st1 = _conv_layer(hcat, nb_pad[3 - i], p['c1W'], None, Nf,
                              2 * oc, Nfp, p['bn1g'], p['bn1b'])
        y2, st = _conv_layer(y1, nb_pad[3 - i], p['c2W'], st1, Nf,
                             oc, Nfp, p['bn2g'], p['bn2b'])
        h = y2

    return _tc_final(h.reshape(CHS[1], npads[0]), st,
                     params['outW'], params['outb'], NS[0])
